# Initial kernel scaffold; baseline (speedup 1.0000x reference)
import jax
import jax.numpy as jnp
from jax.experimental import pallas as pl


def kernel(point_features, point, W1, a_src1, a_dst1, b1, W2, a_src2, a_dst2, b2, Wm, bm):
    raise NotImplementedError("write your pallas kernel here")


# speedup vs baseline: 10.8991x; 10.8991x over previous
import functools

import jax
import jax.numpy as jnp
from jax import lax
from jax.experimental import pallas as pl
from jax.experimental.pallas import tpu as pltpu
from jax.experimental.pallas import tpu_sc as plsc

H = 4
OC = 32
KN = 16
NB = 4
NPT = 2500
NPAD = 2560
TQ = 256
TOT = NB * NPAD
BIG = 1e30


def _knn_lin_body(xq_ref, xk_ref, w_ref, acat_ref, nbr_ref, h_ref, al_ref):
    b = pl.program_id(0)
    q = pl.program_id(1)
    xq = xq_ref[0]
    xk = xk_ref[0]
    h = lax.dot_general(xq, w_ref[...], (((1,), (0,)), ((), ())),
                        preferred_element_type=jnp.float32)
    h_ref[0] = h
    al_ref[0] = lax.dot_general(h, acat_ref[...], (((1,), (0,)), ((), ())),
                                preferred_element_type=jnp.float32,
                                precision=lax.Precision.HIGHEST)
    sqq = jnp.sum(xq * xq, axis=1, keepdims=True)
    sqk = lax.dot_general(jnp.ones((8, xk.shape[1]), jnp.float32), xk * xk,
                          (((1,), (1,)), ((), ())),
                          preferred_element_type=jnp.float32,
                          precision=lax.Precision.HIGHEST)[0:1]
    s = lax.dot_general(xq, xk, (((1,), (1,)), ((), ())),
                        preferred_element_type=jnp.float32)
    val = (sqq + sqk) - 2.0 * s
    kidx = lax.broadcasted_iota(jnp.int32, (TQ, NPAD), 1)
    qidx = q * TQ + lax.broadcasted_iota(jnp.int32, (TQ, NPAD), 0)
    val = jnp.where((kidx == qidx) | (kidx >= NPT), BIG, val)
    cols = []
    for _ in range(KN):
        m = jnp.min(val, axis=1, keepdims=True)
        cand = jnp.where(val == m, kidx, jnp.int32(2 ** 30))
        idx = jnp.min(cand, axis=1, keepdims=True)
        val = jnp.where(kidx == idx, BIG, val)
        cols.append(idx)
    nbr_ref[0] = jnp.concatenate(cols, axis=1) + b * NPAD


def _knn_lin(x, w, acat):
    B, Np, D = x.shape
    grid = (B, Np // TQ)
    return pl.pallas_call(
        _knn_lin_body,
        grid=grid,
        in_specs=[
            pl.BlockSpec((1, TQ, D), lambda b, q: (b, q, 0)),
            pl.BlockSpec((1, Np, D), lambda b, q: (b, 0, 0)),
            pl.BlockSpec((D, 128), lambda b, q: (0, 0)),
            pl.BlockSpec((128, 8), lambda b, q: (0, 0)),
        ],
        out_specs=[
            pl.BlockSpec((1, TQ, KN), lambda b, q: (b, q, 0)),
            pl.BlockSpec((1, TQ, 128), lambda b, q: (b, q, 0)),
            pl.BlockSpec((1, TQ, 8), lambda b, q: (b, q, 0)),
        ],
        out_shape=[
            jax.ShapeDtypeStruct((B, Np, KN), jnp.int32),
            jax.ShapeDtypeStruct((B, Np, 128), jnp.float32),
            jax.ShapeDtypeStruct((B, Np, 8), jnp.float32),
        ],
    )(x, x, w, acat)


_NC = 2
_NS = 16
_NW = _NC * _NS
_NPW = TOT // _NW
_CH = 8
_NCHUNK = _NPW // _CH


def _gat_sc(h2d, al8, nbrf, bias):
    mesh = plsc.VectorSubcoreMesh(core_axis_name="c", subcore_axis_name="s")

    @functools.partial(
        pl.kernel,
        mesh=mesh,
        compiler_params=pltpu.CompilerParams(needs_layout_passes=False),
        out_type=jax.ShapeDtypeStruct((TOT, 128), jnp.float32),
        scratch_types=[
            pltpu.VMEM((TOT * 8,), jnp.float32),
            pltpu.VMEM((128,), jnp.float32),
            pltpu.VMEM((_CH * KN,), jnp.int32),
            pltpu.VMEM((_CH * KN, 128), jnp.float32),
            pltpu.VMEM((_CH, 128), jnp.float32),
            pltpu.SemaphoreType.DMA,
        ],
    )
    def kern(h_hbm, al_hbm, nbr_hbm, b_hbm, out_hbm,
             al_tab, b_tab, idx_v, rows_v, out_v, sem):
        wid = lax.axis_index("s") * _NC + lax.axis_index("c")
        pltpu.sync_copy(al_hbm, al_tab)
        pltpu.sync_copy(b_hbm, b_tab)
        base = wid * _NPW

        def chunk(c, carry):
            nb = base + c * _CH
            pltpu.sync_copy(nbr_hbm.at[pl.ds(nb * KN, _CH * KN)], idx_v)
            pltpu.async_copy(h_hbm.at[idx_v], rows_v, sem).wait()

            def node(p, carry2):
                i = nb + p
                idx16 = idx_v[pl.ds(p * KN, KN)]
                for hh in range(H):
                    src = plsc.load_gather(al_tab, [idx16 * 8 + hh])
                    ivec = jnp.full((KN,), i * 8 + (H + hh), jnp.int32)
                    dst = plsc.load_gather(al_tab, [ivec])
                    e = src + dst
                    e = jnp.where(e > 0.0, e, 0.2 * e)
                    m = jnp.max(e)
                    pe = jnp.exp(e - m)
                    alpha = pe / jnp.sum(pe)
                    for half in range(2):
                        co = hh * OC + half * 16
                        acc = b_tab[pl.ds(co, 16)]
                        for j in range(KN):
                            acc = acc + alpha[j] * rows_v[p * KN + j,
                                                          pl.ds(co, 16)]
                        acc = jnp.where(acc > 0.0, acc, jnp.exp(acc) - 1.0)
                        out_v[p, pl.ds(co, 16)] = acc
                return carry2

            lax.fori_loop(0, _CH, node, 0)
            pltpu.sync_copy(out_v, out_hbm.at[pl.ds(nb, _CH)])
            return carry

        lax.fori_loop(0, _NCHUNK, chunk, 0)

    return kern(h2d, al8, nbrf, bias)


def _mlp_body(x1_ref, x2_ref, w_ref, b_ref, y_ref):
    y = lax.dot_general(x1_ref[...], w_ref[0:128, :], (((1,), (0,)), ((), ())),
                        preferred_element_type=jnp.float32,
                        precision=lax.Precision.HIGHEST)
    y = y + lax.dot_general(x2_ref[...], w_ref[128:256, :],
                            (((1,), (0,)), ((), ())),
                            preferred_element_type=jnp.float32,
                            precision=lax.Precision.HIGHEST)
    y = y + b_ref[...]
    y_ref[...] = jnp.where(y > 0.0, y, jnp.exp(y) - 1.0)


def _mlp(x1, x2, wm, bm):
    T = x1.shape[0]
    return pl.pallas_call(
        _mlp_body,
        grid=(T // TQ,),
        in_specs=[
            pl.BlockSpec((TQ, 128), lambda i: (i, 0)),
            pl.BlockSpec((TQ, 128), lambda i: (i, 0)),
            pl.BlockSpec((256, 128), lambda i: (0, 0)),
            pl.BlockSpec((1, 128), lambda i: (0, 0)),
        ],
        out_specs=pl.BlockSpec((TQ, 128), lambda i: (i, 0)),
        out_shape=jax.ShapeDtypeStruct((T, 128), jnp.float32),
    )(x1, x2, wm, bm.reshape(1, 128))


def _acat(a_src, a_dst):
    eye = jnp.eye(H, dtype=jnp.float32)
    ms = (a_src[:, :, None] * eye[:, None, :]).reshape(H * OC, H)
    md = (a_dst[:, :, None] * eye[:, None, :]).reshape(H * OC, H)
    return jnp.concatenate([ms, md], axis=1)


def kernel(point_features, point, W1, a_src1, a_dst1, b1,
           W2, a_src2, a_dst2, b2, Wm, bm):
    B, NF, Np = point_features.shape
    pf = jnp.concatenate([point_features, point], axis=1)
    pf = jnp.transpose(pf, (0, 2, 1))
    pf = jnp.pad(pf, ((0, 0), (0, NPAD - Np), (0, 0)))

    nbr1, h1, al1 = _knn_lin(pf, W1, _acat(a_src1, a_dst1))
    x1 = _gat_sc(h1.reshape(TOT, 128), al1.reshape(TOT * 8),
                 nbr1.reshape(TOT * KN), b1)
    nbr2, h2, al2 = _knn_lin(x1.reshape(NB, NPAD, 128), W2,
                             _acat(a_src2, a_dst2))
    x2 = _gat_sc(h2.reshape(TOT, 128), al2.reshape(TOT * 8),
                 nbr2.reshape(TOT * KN), b2)
    y = _mlp(x1, x2, Wm, bm)
    y = y.reshape(NB, NPAD, NF)[:, :Np, :]
    return jnp.transpose(y, (0, 2, 1))


# speedup vs baseline: 13.7984x; 1.2660x over previous
import functools

import jax
import jax.numpy as jnp
from jax import lax
from jax.experimental import pallas as pl
from jax.experimental.pallas import tpu as pltpu
from jax.experimental.pallas import tpu_sc as plsc

H = 4
OC = 32
KN = 16
NB = 4
NPT = 2500
NPAD = 2560
TQ = 256
TOT = NB * NPAD
BIG = 1e30


def _knn_lin_body(xq_ref, xk_ref, w_ref, acat_ref, nbr_ref, h_ref, al_ref):
    b = pl.program_id(0)
    q = pl.program_id(1)
    xq = xq_ref[0]
    xk = xk_ref[0]
    h = lax.dot_general(xq, w_ref[...], (((1,), (0,)), ((), ())),
                        preferred_element_type=jnp.float32)
    h_ref[0] = h
    al_ref[0] = lax.dot_general(h, acat_ref[...], (((1,), (0,)), ((), ())),
                                preferred_element_type=jnp.float32,
                                precision=lax.Precision.HIGHEST)
    sqq = jnp.sum(xq * xq, axis=1, keepdims=True)
    sqk = lax.dot_general(jnp.ones((8, xk.shape[1]), jnp.float32), xk * xk,
                          (((1,), (1,)), ((), ())),
                          preferred_element_type=jnp.float32,
                          precision=lax.Precision.HIGHEST)[0:1]
    s = lax.dot_general(xq, xk, (((1,), (1,)), ((), ())),
                        preferred_element_type=jnp.float32)
    val = (sqq + sqk) - 2.0 * s
    kidx = lax.broadcasted_iota(jnp.int32, (TQ, NPAD), 1)
    qidx = q * TQ + lax.broadcasted_iota(jnp.int32, (TQ, NPAD), 0)
    val = jnp.where((kidx == qidx) | (kidx >= NPT), BIG, val)
    cols = []
    for _ in range(KN):
        idx = jnp.argmin(val, axis=1).astype(jnp.int32).reshape(TQ, 1)
        val = jnp.where(kidx == idx, BIG, val)
        cols.append(idx)
    nbr_ref[0] = jnp.concatenate(cols, axis=1) + b * NPAD


def _knn_lin(x, w, acat):
    B, Np, D = x.shape
    grid = (B, Np // TQ)
    return pl.pallas_call(
        _knn_lin_body,
        grid=grid,
        in_specs=[
            pl.BlockSpec((1, TQ, D), lambda b, q: (b, q, 0)),
            pl.BlockSpec((1, Np, D), lambda b, q: (b, 0, 0)),
            pl.BlockSpec((D, 128), lambda b, q: (0, 0)),
            pl.BlockSpec((128, 8), lambda b, q: (0, 0)),
        ],
        out_specs=[
            pl.BlockSpec((1, TQ, KN), lambda b, q: (b, q, 0)),
            pl.BlockSpec((1, TQ, 128), lambda b, q: (b, q, 0)),
            pl.BlockSpec((1, TQ, 8), lambda b, q: (b, q, 0)),
        ],
        out_shape=[
            jax.ShapeDtypeStruct((B, Np, KN), jnp.int32),
            jax.ShapeDtypeStruct((B, Np, 128), jnp.float32),
            jax.ShapeDtypeStruct((B, Np, 8), jnp.float32),
        ],
    )(x, x, w, acat)


_NC = 2
_NS = 16
_NW = _NC * _NS
_NPW = TOT // _NW
_CH = 8
_NCHUNK = _NPW // _CH
_NPAIR = _NCHUNK // 2


def _gat_sc(h2d, al8f, nbrf, bias):
    mesh = plsc.VectorSubcoreMesh(core_axis_name="c", subcore_axis_name="s")

    @functools.partial(
        pl.kernel,
        mesh=mesh,
        compiler_params=pltpu.CompilerParams(needs_layout_passes=False),
        out_type=jax.ShapeDtypeStruct((TOT, 128), jnp.float32),
        scratch_types=[
            pltpu.VMEM((TOT * 8,), jnp.float32),
            pltpu.VMEM((_NPW * KN,), jnp.int32),
            pltpu.VMEM((128,), jnp.float32),
            pltpu.VMEM((2, _CH * KN, 128), jnp.float32),
            pltpu.VMEM((2, _CH, 128), jnp.float32),
        ] + [pltpu.SemaphoreType.DMA] * 4,
    )
    def kern(h_hbm, al_hbm, nbr_hbm, b_hbm, out_hbm,
             al_tab, idx_all, b_tab, rows_v, out_v,
             rsem0, rsem1, osem0, osem1):
        rsem = (rsem0, rsem1)
        osem = (osem0, osem1)
        wid = lax.axis_index("s") * _NC + lax.axis_index("c")
        base = wid * _NPW
        pltpu.sync_copy(al_hbm, al_tab)
        pltpu.sync_copy(nbr_hbm.at[pl.ds(base * KN, _NPW * KN)], idx_all)
        pltpu.sync_copy(b_hbm, b_tab)

        def issue_rows(g, s):
            idx_ref = idx_all.at[pl.ds(g * _CH * KN, _CH * KN)]
            pltpu.async_copy(h_hbm.at[idx_ref], rows_v.at[s], rsem[s])

        def wait_rows(s):
            pltpu.make_async_copy(h_hbm.at[pl.ds(0, _CH * KN)],
                                  rows_v.at[s], rsem[s]).wait()

        def issue_out(g, s):
            pltpu.async_copy(out_v.at[s],
                             out_hbm.at[pl.ds(base + g * _CH, _CH)], osem[s])

        def wait_out(s):
            pltpu.make_async_copy(out_v.at[s], out_hbm.at[pl.ds(0, _CH)],
                                  osem[s]).wait()

        issue_rows(0, 0)
        issue_rows(1, 1)

        def pair(i, carry):
            for s in (0, 1):
                g = 2 * i + s
                wait_rows(s)

                @pl.when(i > 0)
                def _():
                    wait_out(s)

                def node(p, carry2, g=g, s=s):
                    i_node = base + g * _CH + p
                    idx16 = idx_all[pl.ds((g * _CH + p) * KN, KN)]
                    for hh in range(H):
                        src = plsc.load_gather(al_tab, [idx16 * 8 + hh])
                        dvec = jnp.full((KN,), i_node * 8 + (H + hh),
                                        jnp.int32)
                        dst = plsc.load_gather(al_tab, [dvec])
                        e = src + dst
                        e = jnp.where(e > 0.0, e, 0.2 * e)
                        m = jnp.max(e)
                        pe = jnp.exp(e - m)
                        alpha = pe / jnp.sum(pe)
                        for half in range(2):
                            co = hh * OC + half * 16
                            acc = b_tab[pl.ds(co, 16)]
                            for j in range(KN):
                                acc = acc + alpha[j] * rows_v[s, p * KN + j,
                                                              pl.ds(co, 16)]
                            acc = jnp.where(acc > 0.0, acc,
                                            jnp.exp(acc) - 1.0)
                            out_v[s, p, pl.ds(co, 16)] = acc
                    return carry2

                lax.fori_loop(0, _CH, node, 0)
                issue_out(g, s)

                @pl.when(i < _NPAIR - 1)
                def _():
                    issue_rows(g + 2, s)
            return carry

        lax.fori_loop(0, _NPAIR, pair, 0)
        wait_out(0)
        wait_out(1)

    return kern(h2d, al8f, nbrf, bias)


def _mlp_body(x1_ref, x2_ref, w_ref, b_ref, y_ref):
    y = lax.dot_general(x1_ref[...], w_ref[0:128, :], (((1,), (0,)), ((), ())),
                        preferred_element_type=jnp.float32,
                        precision=lax.Precision.HIGHEST)
    y = y + lax.dot_general(x2_ref[...], w_ref[128:256, :],
                            (((1,), (0,)), ((), ())),
                            preferred_element_type=jnp.float32,
                            precision=lax.Precision.HIGHEST)
    y = y + b_ref[...]
    y_ref[...] = jnp.where(y > 0.0, y, jnp.exp(y) - 1.0)


def _mlp(x1, x2, wm, bm):
    T = x1.shape[0]
    return pl.pallas_call(
        _mlp_body,
        grid=(T // TQ,),
        in_specs=[
            pl.BlockSpec((TQ, 128), lambda i: (i, 0)),
            pl.BlockSpec((TQ, 128), lambda i: (i, 0)),
            pl.BlockSpec((256, 128), lambda i: (0, 0)),
            pl.BlockSpec((1, 128), lambda i: (0, 0)),
        ],
        out_specs=pl.BlockSpec((TQ, 128), lambda i: (i, 0)),
        out_shape=jax.ShapeDtypeStruct((T, 128), jnp.float32),
    )(x1, x2, wm, bm.reshape(1, 128))


def _acat(a_src, a_dst):
    eye = jnp.eye(H, dtype=jnp.float32)
    ms = (a_src[:, :, None] * eye[:, None, :]).reshape(H * OC, H)
    md = (a_dst[:, :, None] * eye[:, None, :]).reshape(H * OC, H)
    return jnp.concatenate([ms, md], axis=1)


def kernel(point_features, point, W1, a_src1, a_dst1, b1,
           W2, a_src2, a_dst2, b2, Wm, bm):
    B, NF, Np = point_features.shape
    pf = jnp.concatenate([point_features, point], axis=1)
    pf = jnp.transpose(pf, (0, 2, 1))
    pf = jnp.pad(pf, ((0, 0), (0, NPAD - Np), (0, 0)))

    nbr1, h1, al1 = _knn_lin(pf, W1, _acat(a_src1, a_dst1))
    x1 = _gat_sc(h1.reshape(TOT, 128), al1.reshape(TOT * 8),
                 nbr1.reshape(TOT * KN), b1)
    nbr2, h2, al2 = _knn_lin(x1.reshape(NB, NPAD, 128), W2,
                             _acat(a_src2, a_dst2))
    x2 = _gat_sc(h2.reshape(TOT, 128), al2.reshape(TOT * 8),
                 nbr2.reshape(TOT * KN), b2)
    y = _mlp(x1, x2, Wm, bm)
    y = y.reshape(NB, NPAD, NF)[:, :Np, :]
    return jnp.transpose(y, (0, 2, 1))


# speedup vs baseline: 17.0739x; 1.2374x over previous
import functools

import jax
import jax.numpy as jnp
from jax import lax
from jax.experimental import pallas as pl
from jax.experimental.pallas import tpu as pltpu
from jax.experimental.pallas import tpu_sc as plsc

H = 4
OC = 32
KN = 16
NB = 4
NPT = 2500
NPAD = 2560
TQ = 256
TOT = NB * NPAD
BIG = 1e30


def _knn_lin_body(xq_ref, xk_ref, w_ref, acat_ref, nbr_ref, h_ref, al_ref):
    b = pl.program_id(0)
    q = pl.program_id(1)
    xq = xq_ref[0]
    xk = xk_ref[0]
    h = lax.dot_general(xq, w_ref[...], (((1,), (0,)), ((), ())),
                        preferred_element_type=jnp.float32)
    h_ref[0] = h
    al_ref[0] = lax.dot_general(h, acat_ref[...], (((1,), (0,)), ((), ())),
                                preferred_element_type=jnp.float32,
                                precision=lax.Precision.HIGHEST)
    sqq = jnp.sum(xq * xq, axis=1, keepdims=True)
    sqk = lax.dot_general(jnp.ones((8, xk.shape[1]), jnp.float32), xk * xk,
                          (((1,), (1,)), ((), ())),
                          preferred_element_type=jnp.float32,
                          precision=lax.Precision.HIGHEST)[0:1]
    s = lax.dot_general(xq, xk, (((1,), (1,)), ((), ())),
                        preferred_element_type=jnp.float32)
    val = (sqq + sqk) - 2.0 * s
    kidx = lax.broadcasted_iota(jnp.int32, (TQ, NPAD), 1)
    qidx = q * TQ + lax.broadcasted_iota(jnp.int32, (TQ, NPAD), 0)
    val = jnp.where((kidx == qidx) | (kidx >= NPT), BIG, val)
    cols = []
    for _ in range(KN):
        idx = jnp.argmin(val, axis=1).astype(jnp.int32).reshape(TQ, 1)
        val = jnp.where(kidx == idx, BIG, val)
        cols.append(idx)
    nbr_ref[0] = jnp.concatenate(cols, axis=1) + b * NPAD


def _knn_lin(x, w, acat):
    B, Np, D = x.shape
    grid = (B, Np // TQ)
    return pl.pallas_call(
        _knn_lin_body,
        grid=grid,
        in_specs=[
            pl.BlockSpec((1, TQ, D), lambda b, q: (b, q, 0)),
            pl.BlockSpec((1, Np, D), lambda b, q: (b, 0, 0)),
            pl.BlockSpec((D, 128), lambda b, q: (0, 0)),
            pl.BlockSpec((128, 8), lambda b, q: (0, 0)),
        ],
        out_specs=[
            pl.BlockSpec((1, TQ, KN), lambda b, q: (b, q, 0)),
            pl.BlockSpec((1, TQ, 128), lambda b, q: (b, q, 0)),
            pl.BlockSpec((1, TQ, 8), lambda b, q: (b, q, 0)),
        ],
        out_shape=[
            jax.ShapeDtypeStruct((B, Np, KN), jnp.int32),
            jax.ShapeDtypeStruct((B, Np, 128), jnp.float32),
            jax.ShapeDtypeStruct((B, Np, 8), jnp.float32),
        ],
    )(x, x, w, acat)


_NC = 2
_NS = 16
_NW = _NC * _NS
_CH = 8


def _gat_sc(h2d, al8f, nbrf, bias):
    T = h2d.shape[0]
    npw = T // _NW
    npair = npw // _CH // 2
    mesh = plsc.VectorSubcoreMesh(core_axis_name="c", subcore_axis_name="s")

    @functools.partial(
        pl.kernel,
        mesh=mesh,
        compiler_params=pltpu.CompilerParams(needs_layout_passes=False),
        out_type=jax.ShapeDtypeStruct((T, 128), jnp.float32),
        scratch_types=[
            pltpu.VMEM((T * 8,), jnp.float32),
            pltpu.VMEM((npw * KN,), jnp.int32),
            pltpu.VMEM((128,), jnp.float32),
            pltpu.VMEM((2, _CH * KN, 128), jnp.float32),
            pltpu.VMEM((2, _CH, 128), jnp.float32),
        ] + [pltpu.SemaphoreType.DMA] * 4,
    )
    def kern(h_hbm, al_hbm, nbr_hbm, b_hbm, out_hbm,
             al_tab, idx_all, b_tab, rows_v, out_v,
             rsem0, rsem1, osem0, osem1):
        rsem = (rsem0, rsem1)
        osem = (osem0, osem1)
        wid = lax.axis_index("s") * _NC + lax.axis_index("c")
        base = wid * npw
        pltpu.sync_copy(al_hbm, al_tab)
        pltpu.sync_copy(nbr_hbm.at[pl.ds(base * KN, npw * KN)], idx_all)
        pltpu.sync_copy(b_hbm, b_tab)

        def issue_rows(g, s):
            idx_ref = idx_all.at[pl.ds(g * _CH * KN, _CH * KN)]
            pltpu.async_copy(h_hbm.at[idx_ref], rows_v.at[s], rsem[s])

        def wait_rows(s):
            pltpu.make_async_copy(h_hbm.at[pl.ds(0, _CH * KN)],
                                  rows_v.at[s], rsem[s]).wait()

        def issue_out(g, s):
            pltpu.async_copy(out_v.at[s],
                             out_hbm.at[pl.ds(base + g * _CH, _CH)], osem[s])

        def wait_out(s):
            pltpu.make_async_copy(out_v.at[s], out_hbm.at[pl.ds(0, _CH)],
                                  osem[s]).wait()

        issue_rows(0, 0)
        issue_rows(1, 1)

        def pair(i, carry):
            for s in (0, 1):
                g = 2 * i + s
                wait_rows(s)

                @pl.when(i > 0)
                def _():
                    wait_out(s)

                def node(p, carry2, g=g, s=s):
                    i_node = base + g * _CH + p
                    idx16 = idx_all[pl.ds((g * _CH + p) * KN, KN)]
                    for hh in range(H):
                        src = plsc.load_gather(al_tab, [idx16 * 8 + hh])
                        dvec = jnp.full((KN,), i_node * 8 + (H + hh),
                                        jnp.int32)
                        dst = plsc.load_gather(al_tab, [dvec])
                        e = src + dst
                        e = jnp.where(e > 0.0, e, 0.2 * e)
                        m = jnp.max(e)
                        pe = jnp.exp(e - m)
                        alpha = pe / jnp.sum(pe)
                        for half in range(2):
                            co = hh * OC + half * 16
                            acc = b_tab[pl.ds(co, 16)]
                            for j in range(KN):
                                acc = acc + alpha[j] * rows_v[s, p * KN + j,
                                                              pl.ds(co, 16)]
                            acc = jnp.where(acc > 0.0, acc,
                                            jnp.exp(acc) - 1.0)
                            out_v[s, p, pl.ds(co, 16)] = acc
                    return carry2

                lax.fori_loop(0, _CH, node, 0)
                issue_out(g, s)

                @pl.when(i < npair - 1)
                def _():
                    issue_rows(g + 2, s)
            return carry

        lax.fori_loop(0, npair, pair, 0)
        wait_out(0)
        wait_out(1)

    return kern(h2d, al8f, nbrf, bias)


def _mlp_body(x1_ref, x2_ref, w_ref, b_ref, y_ref):
    y = lax.dot_general(x1_ref[...], w_ref[0:128, :], (((1,), (0,)), ((), ())),
                        preferred_element_type=jnp.float32,
                        precision=lax.Precision.HIGHEST)
    y = y + lax.dot_general(x2_ref[...], w_ref[128:256, :],
                            (((1,), (0,)), ((), ())),
                            preferred_element_type=jnp.float32,
                            precision=lax.Precision.HIGHEST)
    y = y + b_ref[...]
    y_ref[...] = jnp.where(y > 0.0, y, jnp.exp(y) - 1.0)


def _mlp(x1, x2, wm, bm):
    T = x1.shape[0]
    return pl.pallas_call(
        _mlp_body,
        grid=(T // TQ,),
        in_specs=[
            pl.BlockSpec((TQ, 128), lambda i: (i, 0)),
            pl.BlockSpec((TQ, 128), lambda i: (i, 0)),
            pl.BlockSpec((256, 128), lambda i: (0, 0)),
            pl.BlockSpec((1, 128), lambda i: (0, 0)),
        ],
        out_specs=pl.BlockSpec((TQ, 128), lambda i: (i, 0)),
        out_shape=jax.ShapeDtypeStruct((T, 128), jnp.float32),
    )(x1, x2, wm, bm.reshape(1, 128))


def _acat(a_src, a_dst):
    eye = jnp.eye(H, dtype=jnp.float32)
    ms = (a_src[:, :, None] * eye[:, None, :]).reshape(H * OC, H)
    md = (a_dst[:, :, None] * eye[:, None, :]).reshape(H * OC, H)
    return jnp.concatenate([ms, md], axis=1)


def kernel(point_features, point, W1, a_src1, a_dst1, b1,
           W2, a_src2, a_dst2, b2, Wm, bm):
    B, NF, Np = point_features.shape
    pf = jnp.concatenate([point_features, point], axis=1)
    pf = jnp.transpose(pf, (0, 2, 1))
    pf = jnp.pad(pf, ((0, 0), (0, NPAD - Np), (0, 0)))

    acat1 = _acat(a_src1, a_dst1)
    acat2 = _acat(a_src2, a_dst2)
    ys = []
    for b in range(B):
        nbr1, h1, al1 = _knn_lin(pf[b:b + 1], W1, acat1)
        x1 = _gat_sc(h1.reshape(NPAD, 128), al1.reshape(NPAD * 8),
                     nbr1.reshape(NPAD * KN), b1)
        nbr2, h2, al2 = _knn_lin(x1.reshape(1, NPAD, 128), W2, acat2)
        x2 = _gat_sc(h2.reshape(NPAD, 128), al2.reshape(NPAD * 8),
                     nbr2.reshape(NPAD * KN), b2)
        ys.append(_mlp(x1, x2, Wm, bm))
    y = jnp.stack(ys)[:, :Np, :]
    return jnp.transpose(y, (0, 2, 1))


# speedup vs baseline: 17.1761x; 1.0060x over previous
import functools

import jax
import jax.numpy as jnp
from jax import lax
from jax.experimental import pallas as pl
from jax.experimental.pallas import tpu as pltpu
from jax.experimental.pallas import tpu_sc as plsc

H = 4
OC = 32
KN = 16
NB = 4
NPT = 2500
NPAD = 2560
TQ = 256
TOT = NB * NPAD
BIG = 1e30


def _knn_lin_body(xq_ref, xk_ref, w_ref, acat_ref, nbr_ref, h_ref, al_ref):
    b = pl.program_id(0)
    q = pl.program_id(1)
    xq = xq_ref[0]
    xk = xk_ref[0]
    h = lax.dot_general(xq, w_ref[...], (((1,), (0,)), ((), ())),
                        preferred_element_type=jnp.float32)
    h_ref[0] = h
    al_ref[0] = lax.dot_general(h, acat_ref[...], (((1,), (0,)), ((), ())),
                                preferred_element_type=jnp.float32,
                                precision=lax.Precision.HIGHEST)
    sqq = jnp.sum(xq * xq, axis=1, keepdims=True)
    sqk = lax.dot_general(jnp.ones((8, xk.shape[1]), jnp.float32), xk * xk,
                          (((1,), (1,)), ((), ())),
                          preferred_element_type=jnp.float32,
                          precision=lax.Precision.HIGHEST)[0:1]
    s = lax.dot_general(xq, xk, (((1,), (1,)), ((), ())),
                        preferred_element_type=jnp.float32)
    val = (sqq + sqk) - 2.0 * s
    kidx = lax.broadcasted_iota(jnp.int32, (TQ, NPAD), 1)
    qidx = q * TQ + lax.broadcasted_iota(jnp.int32, (TQ, NPAD), 0)
    val = jnp.where((kidx == qidx) | (kidx >= NPT), BIG, val)
    cols = []
    for _ in range(KN):
        idx = jnp.argmin(val, axis=1).astype(jnp.int32).reshape(TQ, 1)
        val = jnp.where(kidx == idx, BIG, val)
        cols.append(idx)
    nbr_ref[0] = jnp.concatenate(cols, axis=1) + b * NPAD


def _knn_lin(x, w, acat):
    B, Np, D = x.shape
    grid = (B, Np // TQ)
    return pl.pallas_call(
        _knn_lin_body,
        grid=grid,
        in_specs=[
            pl.BlockSpec((1, TQ, D), lambda b, q: (b, q, 0)),
            pl.BlockSpec((1, Np, D), lambda b, q: (b, 0, 0)),
            pl.BlockSpec((D, 128), lambda b, q: (0, 0)),
            pl.BlockSpec((128, 8), lambda b, q: (0, 0)),
        ],
        out_specs=[
            pl.BlockSpec((1, TQ, KN), lambda b, q: (b, q, 0)),
            pl.BlockSpec((1, TQ, 128), lambda b, q: (b, q, 0)),
            pl.BlockSpec((1, TQ, 8), lambda b, q: (b, q, 0)),
        ],
        out_shape=[
            jax.ShapeDtypeStruct((B, Np, KN), jnp.int32),
            jax.ShapeDtypeStruct((B, Np, 128), jnp.float32),
            jax.ShapeDtypeStruct((B, Np, 8), jnp.float32),
        ],
    )(x, x, w, acat)


_NC = 2
_NS = 16
_NW = _NC * _NS
_CH = 8


def _gat_sc(h2d, al8f, nbrf, bias):
    T = h2d.shape[0]
    npw = T // _NW
    npair = npw // _CH // 2
    mesh = plsc.VectorSubcoreMesh(core_axis_name="c", subcore_axis_name="s")

    @functools.partial(
        pl.kernel,
        mesh=mesh,
        compiler_params=pltpu.CompilerParams(needs_layout_passes=False),
        out_type=jax.ShapeDtypeStruct((T, 128), jnp.float32),
        scratch_types=[
            pltpu.VMEM((T * 8,), jnp.float32),
            pltpu.VMEM((npw * KN,), jnp.int32),
            pltpu.VMEM((128,), jnp.float32),
            pltpu.VMEM((2, _CH * KN, 128), jnp.float32),
            pltpu.VMEM((2, _CH, 128), jnp.float32),
        ] + [pltpu.SemaphoreType.DMA] * 4,
    )
    def kern(h_hbm, al_hbm, nbr_hbm, b_hbm, out_hbm,
             al_tab, idx_all, b_tab, rows_v, out_v,
             rsem0, rsem1, osem0, osem1):
        rsem = (rsem0, rsem1)
        osem = (osem0, osem1)
        wid = lax.axis_index("s") * _NC + lax.axis_index("c")
        base = wid * npw
        pltpu.sync_copy(al_hbm, al_tab)
        pltpu.sync_copy(nbr_hbm.at[pl.ds(base * KN, npw * KN)], idx_all)
        pltpu.sync_copy(b_hbm, b_tab)

        def issue_rows(g, s):
            idx_ref = idx_all.at[pl.ds(g * _CH * KN, _CH * KN)]
            pltpu.async_copy(h_hbm.at[idx_ref], rows_v.at[s], rsem[s])

        def wait_rows(s):
            pltpu.make_async_copy(h_hbm.at[pl.ds(0, _CH * KN)],
                                  rows_v.at[s], rsem[s]).wait()

        def issue_out(g, s):
            pltpu.async_copy(out_v.at[s],
                             out_hbm.at[pl.ds(base + g * _CH, _CH)], osem[s])

        def wait_out(s):
            pltpu.make_async_copy(out_v.at[s], out_hbm.at[pl.ds(0, _CH)],
                                  osem[s]).wait()

        issue_rows(0, 0)
        issue_rows(1, 1)

        def pair(i, carry):
            for s in (0, 1):
                g = 2 * i + s
                wait_rows(s)

                @pl.when(i > 0)
                def _():
                    wait_out(s)

                def node(p, carry2, g=g, s=s):
                    i_node = base + g * _CH + p
                    idx16 = idx_all[pl.ds((g * _CH + p) * KN, KN)]
                    for hh in range(H):
                        src = plsc.load_gather(al_tab, [idx16 * 8 + hh])
                        dvec = jnp.full((KN,), i_node * 8 + (H + hh),
                                        jnp.int32)
                        dst = plsc.load_gather(al_tab, [dvec])
                        e = src + dst
                        e = jnp.where(e > 0.0, e, 0.2 * e)
                        m = jnp.max(e)
                        pe = jnp.exp(e - m)
                        alpha = pe / jnp.sum(pe)
                        co = hh * OC
                        acc0 = b_tab[pl.ds(co, 16)]
                        acc1 = b_tab[pl.ds(co + 16, 16)]
                        for j in range(KN):
                            aj = alpha[j]
                            acc0 = acc0 + aj * rows_v[s, p * KN + j,
                                                      pl.ds(co, 16)]
                            acc1 = acc1 + aj * rows_v[s, p * KN + j,
                                                      pl.ds(co + 16, 16)]
                        acc0 = jnp.where(acc0 > 0.0, acc0,
                                         jnp.exp(acc0) - 1.0)
                        acc1 = jnp.where(acc1 > 0.0, acc1,
                                         jnp.exp(acc1) - 1.0)
                        out_v[s, p, pl.ds(co, 16)] = acc0
                        out_v[s, p, pl.ds(co + 16, 16)] = acc1
                    return carry2

                lax.fori_loop(0, _CH, node, 0)
                issue_out(g, s)

                @pl.when(i < npair - 1)
                def _():
                    issue_rows(g + 2, s)
            return carry

        lax.fori_loop(0, npair, pair, 0)
        wait_out(0)
        wait_out(1)

    return kern(h2d, al8f, nbrf, bias)


def _mlp_body(x1_ref, x2_ref, w_ref, b_ref, y_ref):
    y = lax.dot_general(x1_ref[...], w_ref[0:128, :], (((1,), (0,)), ((), ())),
                        preferred_element_type=jnp.float32,
                        precision=lax.Precision.HIGHEST)
    y = y + lax.dot_general(x2_ref[...], w_ref[128:256, :],
                            (((1,), (0,)), ((), ())),
                            preferred_element_type=jnp.float32,
                            precision=lax.Precision.HIGHEST)
    y = y + b_ref[...]
    y_ref[...] = jnp.where(y > 0.0, y, jnp.exp(y) - 1.0)


def _mlp(x1, x2, wm, bm):
    T = x1.shape[0]
    return pl.pallas_call(
        _mlp_body,
        grid=(T // TQ,),
        in_specs=[
            pl.BlockSpec((TQ, 128), lambda i: (i, 0)),
            pl.BlockSpec((TQ, 128), lambda i: (i, 0)),
            pl.BlockSpec((256, 128), lambda i: (0, 0)),
            pl.BlockSpec((1, 128), lambda i: (0, 0)),
        ],
        out_specs=pl.BlockSpec((TQ, 128), lambda i: (i, 0)),
        out_shape=jax.ShapeDtypeStruct((T, 128), jnp.float32),
    )(x1, x2, wm, bm.reshape(1, 128))


def _acat(a_src, a_dst):
    eye = jnp.eye(H, dtype=jnp.float32)
    ms = (a_src[:, :, None] * eye[:, None, :]).reshape(H * OC, H)
    md = (a_dst[:, :, None] * eye[:, None, :]).reshape(H * OC, H)
    return jnp.concatenate([ms, md], axis=1)


def kernel(point_features, point, W1, a_src1, a_dst1, b1,
           W2, a_src2, a_dst2, b2, Wm, bm):
    B, NF, Np = point_features.shape
    pf = jnp.concatenate([point_features, point], axis=1)
    pf = jnp.transpose(pf, (0, 2, 1))
    pf = jnp.pad(pf, ((0, 0), (0, NPAD - Np), (0, 0)))

    acat1 = _acat(a_src1, a_dst1)
    acat2 = _acat(a_src2, a_dst2)
    ys = []
    for b in range(B):
        nbr1, h1, al1 = _knn_lin(pf[b:b + 1], W1, acat1)
        x1 = _gat_sc(h1.reshape(NPAD, 128), al1.reshape(NPAD * 8),
                     nbr1.reshape(NPAD * KN), b1)
        nbr2, h2, al2 = _knn_lin(x1.reshape(1, NPAD, 128), W2, acat2)
        x2 = _gat_sc(h2.reshape(NPAD, 128), al2.reshape(NPAD * 8),
                     nbr2.reshape(NPAD * KN), b2)
        ys.append(_mlp(x1, x2, Wm, bm))
    y = jnp.stack(ys)[:, :Np, :]
    return jnp.transpose(y, (0, 2, 1))


# speedup vs baseline: 17.1901x; 1.0008x over previous
import functools

import jax
import jax.numpy as jnp
from jax import lax
from jax.experimental import pallas as pl
from jax.experimental.pallas import tpu as pltpu
from jax.experimental.pallas import tpu_sc as plsc

H = 4
OC = 32
KN = 16
NB = 4
NPT = 2500
NPAD = 2560
TQ = 256
TOT = NB * NPAD
BIG = 1e30


def _knn_lin_body(xq_ref, xk_ref, w_ref, acat_ref, nbr_ref, h_ref, al_ref):
    b = pl.program_id(0)
    q = pl.program_id(1)
    xq = xq_ref[0]
    xk = xk_ref[0]
    h = lax.dot_general(xq, w_ref[...], (((1,), (0,)), ((), ())),
                        preferred_element_type=jnp.float32)
    h_ref[0] = h
    al_ref[0] = lax.dot_general(h, acat_ref[...], (((1,), (0,)), ((), ())),
                                preferred_element_type=jnp.float32,
                                precision=lax.Precision.HIGHEST)
    sqq = jnp.sum(xq * xq, axis=1, keepdims=True)
    sqk = lax.dot_general(jnp.ones((8, xk.shape[1]), jnp.float32), xk * xk,
                          (((1,), (1,)), ((), ())),
                          preferred_element_type=jnp.float32,
                          precision=lax.Precision.HIGHEST)[0:1]
    s = lax.dot_general(xq, xk, (((1,), (1,)), ((), ())),
                        preferred_element_type=jnp.float32)
    val = (sqq + sqk) - 2.0 * s
    kidx = lax.broadcasted_iota(jnp.int32, (TQ, NPAD), 1)
    qidx = q * TQ + lax.broadcasted_iota(jnp.int32, (TQ, NPAD), 0)
    val = jnp.where((kidx == qidx) | (kidx >= NPT), BIG, val)
    cols = []
    for _ in range(KN):
        idx = jnp.argmin(val, axis=1).astype(jnp.int32).reshape(TQ, 1)
        val = jnp.where(kidx == idx, BIG, val)
        cols.append(idx)
    nbr_ref[0] = jnp.concatenate(cols, axis=1) + b * NPAD


def _knn_lin(x, w, acat):
    B, Np, D = x.shape
    grid = (B, Np // TQ)
    return pl.pallas_call(
        _knn_lin_body,
        grid=grid,
        in_specs=[
            pl.BlockSpec((1, TQ, D), lambda b, q: (b, q, 0)),
            pl.BlockSpec((1, Np, D), lambda b, q: (b, 0, 0)),
            pl.BlockSpec((D, 128), lambda b, q: (0, 0)),
            pl.BlockSpec((128, 8), lambda b, q: (0, 0)),
        ],
        out_specs=[
            pl.BlockSpec((1, TQ, KN), lambda b, q: (b, q, 0)),
            pl.BlockSpec((1, TQ, 128), lambda b, q: (b, q, 0)),
            pl.BlockSpec((1, TQ, 8), lambda b, q: (b, q, 0)),
        ],
        out_shape=[
            jax.ShapeDtypeStruct((B, Np, KN), jnp.int32),
            jax.ShapeDtypeStruct((B, Np, 128), jnp.float32),
            jax.ShapeDtypeStruct((B, Np, 8), jnp.float32),
        ],
    )(x, x, w, acat)


_NC = 2
_NS = 16
_NW = _NC * _NS
_CH = 8


def _gat_sc(h2d, al8f, nbrf, bias):
    T = h2d.shape[0]
    npw = T // _NW
    npair = npw // _CH // 2
    mesh = plsc.VectorSubcoreMesh(core_axis_name="c", subcore_axis_name="s")

    @functools.partial(
        pl.kernel,
        mesh=mesh,
        compiler_params=pltpu.CompilerParams(needs_layout_passes=False),
        out_type=jax.ShapeDtypeStruct((T, 128), jnp.float32),
        scratch_types=[
            pltpu.VMEM((T * 8,), jnp.float32),
            pltpu.VMEM((npw * KN,), jnp.int32),
            pltpu.VMEM((128,), jnp.float32),
            pltpu.VMEM((2, _CH * KN, 128), jnp.float32),
            pltpu.VMEM((2, _CH, 128), jnp.float32),
        ] + [pltpu.SemaphoreType.DMA] * 4,
    )
    def kern(h_hbm, al_hbm, nbr_hbm, b_hbm, out_hbm,
             al_tab, idx_all, b_tab, rows_v, out_v,
             rsem0, rsem1, osem0, osem1):
        rsem = (rsem0, rsem1)
        osem = (osem0, osem1)
        wid = lax.axis_index("s") * _NC + lax.axis_index("c")
        base = wid * npw
        pltpu.sync_copy(al_hbm, al_tab)
        pltpu.sync_copy(nbr_hbm.at[pl.ds(base * KN, npw * KN)], idx_all)
        pltpu.sync_copy(b_hbm, b_tab)

        def issue_rows(g, s):
            idx_ref = idx_all.at[pl.ds(g * _CH * KN, _CH * KN)]
            pltpu.async_copy(h_hbm.at[idx_ref], rows_v.at[s], rsem[s])

        def wait_rows(s):
            pltpu.make_async_copy(h_hbm.at[pl.ds(0, _CH * KN)],
                                  rows_v.at[s], rsem[s]).wait()

        def issue_out(g, s):
            pltpu.async_copy(out_v.at[s],
                             out_hbm.at[pl.ds(base + g * _CH, _CH)], osem[s])

        def wait_out(s):
            pltpu.make_async_copy(out_v.at[s], out_hbm.at[pl.ds(0, _CH)],
                                  osem[s]).wait()

        issue_rows(0, 0)
        issue_rows(1, 1)

        def pair(i, carry):
            for s in (0, 1):
                g = 2 * i + s
                wait_rows(s)

                @pl.when(i > 0)
                def _():
                    wait_out(s)

                def node(p, carry2, g=g, s=s):
                    i_node = base + g * _CH + p
                    idx16 = idx_all[pl.ds((g * _CH + p) * KN, KN)]
                    for hh in range(H):
                        src = plsc.load_gather(al_tab, [idx16 * 8 + hh])
                        dvec = jnp.full((KN,), i_node * 8 + (H + hh),
                                        jnp.int32)
                        dst = plsc.load_gather(al_tab, [dvec])
                        e = src + dst
                        e = jnp.where(e > 0.0, e, 0.2 * e)
                        m = jnp.max(e)
                        pe = jnp.exp(e - m)
                        alpha = pe / jnp.sum(pe)
                        co = hh * OC
                        a0 = alpha[0]
                        acc0 = a0 * rows_v[s, p * KN, pl.ds(co, 16)]
                        acc1 = a0 * rows_v[s, p * KN, pl.ds(co + 16, 16)]
                        for j in range(1, KN):
                            aj = alpha[j]
                            acc0 = acc0 + aj * rows_v[s, p * KN + j,
                                                      pl.ds(co, 16)]
                            acc1 = acc1 + aj * rows_v[s, p * KN + j,
                                                      pl.ds(co + 16, 16)]
                        acc0 = acc0 + b_tab[pl.ds(co, 16)]
                        acc1 = acc1 + b_tab[pl.ds(co + 16, 16)]
                        acc0 = jnp.where(acc0 > 0.0, acc0,
                                         jnp.exp(acc0) - 1.0)
                        acc1 = jnp.where(acc1 > 0.0, acc1,
                                         jnp.exp(acc1) - 1.0)
                        out_v[s, p, pl.ds(co, 16)] = acc0
                        out_v[s, p, pl.ds(co + 16, 16)] = acc1
                    return carry2

                lax.fori_loop(0, _CH, node, 0)
                issue_out(g, s)

                @pl.when(i < npair - 1)
                def _():
                    issue_rows(g + 2, s)
            return carry

        lax.fori_loop(0, npair, pair, 0)
        wait_out(0)
        wait_out(1)

    return kern(h2d, al8f, nbrf, bias)


def _mlp_body(x1_ref, x2_ref, w_ref, b_ref, y_ref):
    y = lax.dot_general(x1_ref[...], w_ref[0:128, :], (((1,), (0,)), ((), ())),
                        preferred_element_type=jnp.float32,
                        precision=lax.Precision.HIGHEST)
    y = y + lax.dot_general(x2_ref[...], w_ref[128:256, :],
                            (((1,), (0,)), ((), ())),
                            preferred_element_type=jnp.float32,
                            precision=lax.Precision.HIGHEST)
    y = y + b_ref[...]
    y_ref[...] = jnp.where(y > 0.0, y, jnp.exp(y) - 1.0)


def _mlp(x1, x2, wm, bm):
    T = x1.shape[0]
    return pl.pallas_call(
        _mlp_body,
        grid=(T // TQ,),
        in_specs=[
            pl.BlockSpec((TQ, 128), lambda i: (i, 0)),
            pl.BlockSpec((TQ, 128), lambda i: (i, 0)),
            pl.BlockSpec((256, 128), lambda i: (0, 0)),
            pl.BlockSpec((1, 128), lambda i: (0, 0)),
        ],
        out_specs=pl.BlockSpec((TQ, 128), lambda i: (i, 0)),
        out_shape=jax.ShapeDtypeStruct((T, 128), jnp.float32),
    )(x1, x2, wm, bm.reshape(1, 128))


def _acat(a_src, a_dst):
    eye = jnp.eye(H, dtype=jnp.float32)
    ms = (a_src[:, :, None] * eye[:, None, :]).reshape(H * OC, H)
    md = (a_dst[:, :, None] * eye[:, None, :]).reshape(H * OC, H)
    return jnp.concatenate([ms, md], axis=1)


def kernel(point_features, point, W1, a_src1, a_dst1, b1,
           W2, a_src2, a_dst2, b2, Wm, bm):
    B, NF, Np = point_features.shape
    pf = jnp.concatenate([point_features, point], axis=1)
    pf = jnp.transpose(pf, (0, 2, 1))
    pf = jnp.pad(pf, ((0, 0), (0, NPAD - Np), (0, 0)))

    acat1 = _acat(a_src1, a_dst1)
    acat2 = _acat(a_src2, a_dst2)
    ys = []
    for b in range(B):
        nbr1, h1, al1 = _knn_lin(pf[b:b + 1], W1, acat1)
        x1 = _gat_sc(h1.reshape(NPAD, 128), al1.reshape(NPAD * 8),
                     nbr1.reshape(NPAD * KN), b1)
        nbr2, h2, al2 = _knn_lin(x1.reshape(1, NPAD, 128), W2, acat2)
        x2 = _gat_sc(h2.reshape(NPAD, 128), al2.reshape(NPAD * 8),
                     nbr2.reshape(NPAD * KN), b2)
        ys.append(_mlp(x1, x2, Wm, bm))
    y = jnp.stack(ys)[:, :Np, :]
    return jnp.transpose(y, (0, 2, 1))
